# baseline (speedup 1.0000x reference)
import functools

import jax
import jax.numpy as jnp
from jax import lax
from jax.experimental import pallas as pl
from jax.experimental.pallas import tpu as pltpu
from jax.experimental.pallas import tpu_sc as plsc

B, D, C = 16384, 512, 512


_BB = 1024


def _winner_body(x_ref, w_ref, out_ref):
    x = x_ref[...]
    w = w_ref[...]
    xw = lax.dot_general(
        x, w, (((1,), (1,)), ((), ())),
        preferred_element_type=jnp.float32,
    )
    x2 = jnp.sum(x * x, axis=1, keepdims=True)
    w2 = jnp.sum(w * w, axis=1)[None, :]
    d2 = jnp.maximum(x2 + w2 - 2.0 * xw, 1e-12)
    out_ref[...] = jnp.argmin(d2, axis=1).astype(jnp.int32)


def _winner_call(x, W, row_off, rows):
    grid = rows // _BB
    blk_off = row_off // _BB
    return pl.pallas_call(
        _winner_body,
        grid=(grid,),
        in_specs=[
            pl.BlockSpec((_BB, D), lambda i, o=blk_off: (i + o, 0)),
            pl.BlockSpec((C, D), lambda i: (0, 0)),
        ],
        out_specs=pl.BlockSpec((_BB,), lambda i: (i,)),
        out_shape=jax.ShapeDtypeStruct((rows,), jnp.int32),
    )(x, W)


_info = plsc.get_sparse_core_info()
_NC, _NS = _info.num_cores, _info.num_subcores
_NW = _NC * _NS
_CHUNK = 64


def _make_gather_body(bpw, offset):
    nchunk = bpw // _CHUNK

    def _gather_body(cw_hbm, idx_hbm, out_hbm, idx_v, rows_v, gsem,
                     wsem0, wsem1):
        cid = lax.axis_index("c")
        sid = lax.axis_index("s")
        wid = sid * _NC + cid
        base = wid * bpw
        pltpu.async_copy(idx_hbm.at[pl.ds(base, bpw)], idx_v, gsem).wait()
        wsems = (wsem0, wsem1)

        def write_copy(j):
            return pltpu.make_async_copy(
                rows_v.at[j % 2],
                out_hbm.at[pl.ds(offset + base + j * _CHUNK, _CHUNK)],
                wsems[j % 2])

        for j in range(nchunk):
            buf = j % 2
            if j >= 2:
                write_copy(j - 2).wait()
            pltpu.async_copy(
                cw_hbm.at[idx_v.at[pl.ds(j * _CHUNK, _CHUNK)]],
                rows_v.at[buf], gsem).wait()
            write_copy(j).start()
        for j in range(max(nchunk - 2, 0), nchunk):
            write_copy(j).wait()

    return _gather_body


def _gather_call(c_w, winner, preds_ref, offset):
    rows = winner.shape[0]
    bpw = rows // _NW
    mesh = plsc.VectorSubcoreMesh(core_axis_name="c", subcore_axis_name="s")
    k = functools.partial(
        pl.kernel,
        mesh=mesh,
        scratch_types=[
            pltpu.VMEM((bpw,), jnp.int32),
            pltpu.VMEM((2, _CHUNK, D), jnp.int32),
            pltpu.SemaphoreType.DMA,
            pltpu.SemaphoreType.DMA,
            pltpu.SemaphoreType.DMA,
        ],
    )(_make_gather_body(bpw, offset))
    k(c_w, winner, preds_ref)


_K = 8


def kernel(x, y, W, c_w):
    bc = B // _K
    preds_ref = jax.new_ref(lax.empty((B, D), jnp.int32))
    for k in range(_K):
        wk = _winner_call(x, W, k * bc, bc)
        _gather_call(c_w, wk, preds_ref, k * bc)
    return preds_ref[...]


# speedup vs baseline: 1.0378x; 1.0378x over previous
import functools

import jax
import jax.numpy as jnp
from jax import lax
from jax.experimental import pallas as pl
from jax.experimental.pallas import tpu as pltpu
from jax.experimental.pallas import tpu_sc as plsc

B, D, C = 16384, 512, 512


_BB = 1024


def _w2_body(w_ref, out_ref):
    w = w_ref[...]
    out_ref[...] = jnp.sum(w * w, axis=1)[None, :]


def _w2_call(W):
    return pl.pallas_call(
        _w2_body,
        out_shape=jax.ShapeDtypeStruct((1, C), jnp.float32),
    )(W)


def _winner_body(x_ref, w_ref, w2_ref, out_ref):
    x = x_ref[...]
    w = w_ref[...]
    xw = lax.dot_general(
        x, w, (((1,), (1,)), ((), ())),
        preferred_element_type=jnp.float32,
    )
    x2 = jnp.sum(x * x, axis=1, keepdims=True)
    d2 = jnp.maximum(x2 + w2_ref[...] - 2.0 * xw, 1e-12)
    out_ref[...] = jnp.argmin(d2, axis=1).astype(jnp.int32)


def _winner_call(x, W, w2, row_off, rows):
    grid = rows // _BB
    blk_off = row_off // _BB
    return pl.pallas_call(
        _winner_body,
        grid=(grid,),
        in_specs=[
            pl.BlockSpec((_BB, D), lambda i, o=blk_off: (i + o, 0)),
            pl.BlockSpec((C, D), lambda i: (0, 0)),
            pl.BlockSpec((1, C), lambda i: (0, 0)),
        ],
        out_specs=pl.BlockSpec((_BB,), lambda i: (i,)),
        out_shape=jax.ShapeDtypeStruct((rows,), jnp.int32),
    )(x, W, w2)


_info = plsc.get_sparse_core_info()
_NC, _NS = _info.num_cores, _info.num_subcores
_NW = _NC * _NS
_CHUNK = 64


def _make_gather_body(bpw, offset):
    nchunk = bpw // _CHUNK

    def _gather_body(cw_hbm, idx_hbm, out_hbm, idx_v, rows_v, gsem,
                     wsem0, wsem1):
        cid = lax.axis_index("c")
        sid = lax.axis_index("s")
        wid = sid * _NC + cid
        base = wid * bpw
        pltpu.async_copy(idx_hbm.at[pl.ds(base, bpw)], idx_v, gsem).wait()
        wsems = (wsem0, wsem1)

        def write_copy(j):
            return pltpu.make_async_copy(
                rows_v.at[j % 2],
                out_hbm.at[pl.ds(offset + base + j * _CHUNK, _CHUNK)],
                wsems[j % 2])

        for j in range(nchunk):
            buf = j % 2
            if j >= 2:
                write_copy(j - 2).wait()
            pltpu.async_copy(
                cw_hbm.at[idx_v.at[pl.ds(j * _CHUNK, _CHUNK)]],
                rows_v.at[buf], gsem).wait()
            write_copy(j).start()
        for j in range(max(nchunk - 2, 0), nchunk):
            write_copy(j).wait()

    return _gather_body


def _gather_call(c_w, winner, preds_ref, offset):
    rows = winner.shape[0]
    bpw = rows // _NW
    mesh = plsc.VectorSubcoreMesh(core_axis_name="c", subcore_axis_name="s")
    k = functools.partial(
        pl.kernel,
        mesh=mesh,
        scratch_types=[
            pltpu.VMEM((bpw,), jnp.int32),
            pltpu.VMEM((2, _CHUNK, D), jnp.int32),
            pltpu.SemaphoreType.DMA,
            pltpu.SemaphoreType.DMA,
            pltpu.SemaphoreType.DMA,
        ],
    )(_make_gather_body(bpw, offset))
    k(c_w, winner, preds_ref)


_K = 4


def kernel(x, y, W, c_w):
    bc = B // _K
    preds_ref = jax.new_ref(lax.empty((B, D), jnp.int32))
    w2 = _w2_call(W)
    for k in range(_K):
        wk = _winner_call(x, W, w2, k * bc, bc)
        _gather_call(c_w, wk, preds_ref, k * bc)
    return preds_ref[...]


# speedup vs baseline: 1.0685x; 1.0295x over previous
import functools

import jax
import jax.numpy as jnp
from jax import lax
from jax.experimental import pallas as pl
from jax.experimental.pallas import tpu as pltpu
from jax.experimental.pallas import tpu_sc as plsc

B, D, C = 16384, 512, 512


_BB = 1024


def _winner_body(x_ref, w_ref, out_ref):
    x = x_ref[...]
    w = w_ref[...]
    xw = lax.dot_general(
        x, w, (((1,), (1,)), ((), ())),
        preferred_element_type=jnp.float32,
    )
    x2 = jnp.sum(x * x, axis=1, keepdims=True)
    w2 = jnp.sum(w * w, axis=1)[None, :]
    d2 = jnp.maximum(x2 + w2 - 2.0 * xw, 1e-12)
    out_ref[...] = jnp.argmin(d2, axis=1).astype(jnp.int32)


def _winner_call(x, W, row_off, rows):
    grid = rows // _BB
    blk_off = row_off // _BB
    return pl.pallas_call(
        _winner_body,
        grid=(grid,),
        in_specs=[
            pl.BlockSpec((_BB, D), lambda i, o=blk_off: (i + o, 0)),
            pl.BlockSpec((C, D), lambda i: (0, 0)),
        ],
        out_specs=pl.BlockSpec((_BB,), lambda i: (i,)),
        out_shape=jax.ShapeDtypeStruct((rows,), jnp.int32),
    )(x, W)


_info = plsc.get_sparse_core_info()
_NC, _NS = _info.num_cores, _info.num_subcores
_NW = _NC * _NS
_CHUNK = 64


def _make_gather_body(bpw, offset):
    nchunk = bpw // _CHUNK

    def _gather_body(cw_hbm, idx_hbm, out_hbm, idx_v, rows_v, gsem,
                     wsem0, wsem1):
        cid = lax.axis_index("c")
        sid = lax.axis_index("s")
        wid = sid * _NC + cid
        base = wid * bpw
        pltpu.async_copy(idx_hbm.at[pl.ds(base, bpw)], idx_v, gsem).wait()
        wsems = (wsem0, wsem1)

        def write_copy(j):
            return pltpu.make_async_copy(
                rows_v.at[j % 2],
                out_hbm.at[pl.ds(offset + base + j * _CHUNK, _CHUNK)],
                wsems[j % 2])

        for j in range(nchunk):
            buf = j % 2
            if j >= 2:
                write_copy(j - 2).wait()
            pltpu.async_copy(
                cw_hbm.at[idx_v.at[pl.ds(j * _CHUNK, _CHUNK)]],
                rows_v.at[buf], gsem).wait()
            write_copy(j).start()
        for j in range(max(nchunk - 2, 0), nchunk):
            write_copy(j).wait()

    return _gather_body


def _gather_call(c_w, winner, preds_ref, offset):
    rows = winner.shape[0]
    bpw = rows // _NW
    mesh = plsc.VectorSubcoreMesh(core_axis_name="c", subcore_axis_name="s")
    k = functools.partial(
        pl.kernel,
        mesh=mesh,
        scratch_types=[
            pltpu.VMEM((bpw,), jnp.int32),
            pltpu.VMEM((2, _CHUNK, D), jnp.int32),
            pltpu.SemaphoreType.DMA,
            pltpu.SemaphoreType.DMA,
            pltpu.SemaphoreType.DMA,
        ],
    )(_make_gather_body(bpw, offset))
    k(c_w, winner, preds_ref)


_K = 4


def kernel(x, y, W, c_w):
    bc = B // _K
    preds_ref = jax.new_ref(lax.empty((B, D), jnp.int32))
    for k in range(_K):
        wk = _winner_call(x, W, k * bc, bc)
        _gather_call(c_w, wk, preds_ref, k * bc)
    return preds_ref[...]
